# Initial kernel scaffold; baseline (speedup 1.0000x reference)
import jax
import jax.numpy as jnp
from jax.experimental import pallas as pl


def kernel(hidden_states, token_ids, tables, qn_g, qn_b, mn_g, mn_b, Wq, Wk, Wv, Wo, bo, on_g, on_b, conv_w, conv_b):
    raise NotImplementedError("write your pallas kernel here")


# speedup vs baseline: 4.4376x; 4.4376x over previous
import functools

import numpy as np
import jax
import jax.numpy as jnp
from jax import lax
from jax.experimental import pallas as pl
from jax.experimental.pallas import tpu as pltpu
from jax.experimental.pallas import tpu_sc as plsc

_NUM_HEADS = 4
_TABLE = 50000
_EMB = 256
_HID = 1536
_KSIZE = 4
_B, _S = 4, 2048
_NPARTS = 8
_ROWS = _B * _S
_CONCAT = _EMB * _NPARTS

_mrng = np.random.RandomState(42)
_MULTS_NP = [_mrng.randint(2, 2 ** 31, size=_NUM_HEADS) for _ in range(2)]
_M = [int(_MULTS_NP[j // _NUM_HEADS][j % _NUM_HEADS]) for j in range(_NPARTS)]
_R16 = (1 << 16) % _TABLE

_NW = 32
_CH = 128
_NCH = _S // _CH
_NBUF = 3


def _sc_body(tok_hbm, table_hbm, out_hbm, tbuf, idxbuf, rows,
             g0, g1, g2, w0, w1, w2):
    gs = (g0, g1, g2)
    ws = (w0, w1, w2)
    cid = lax.axis_index("c")
    sid = lax.axis_index("s")
    wid = (sid * 2 + cid).astype(jnp.int32)
    j = lax.div(wid, jnp.int32(_B))
    b = lax.rem(wid, jnp.int32(_B))

    tbuf[pl.ds(0, 16)] = jnp.zeros((16,), jnp.int32)
    pltpu.sync_copy(tok_hbm.at[pl.ds(b * _S, _S)], tbuf.at[pl.ds(16, _S)])

    m_lo = jnp.uint32(0)
    m_hi = jnp.uint32(0)
    for jj in range(_NPARTS):
        sel = j == jj
        m_lo = jnp.where(sel, jnp.uint32(_M[jj] & 0xFFFF), m_lo)
        m_hi = jnp.where(sel, jnp.uint32(_M[jj] >> 16), m_hi)
    sel2 = jnp.where(j < _NUM_HEADS, jnp.uint32(1), jnp.uint32(0))

    T = jnp.uint32(_TABLE)
    R = jnp.uint32(_R16)

    def mm(h):
        a = lax.rem(h * m_lo, T)
        c = lax.rem(h * m_hi, T)
        return lax.rem(c * R + a, T)

    def hash_body(_, pos):
        t0 = tbuf[pl.ds(16 + pos, 16)].astype(jnp.uint32)
        t1 = tbuf[pl.ds(15 + pos, 16)].astype(jnp.uint32)
        t2 = tbuf[pl.ds(14 + pos, 16)].astype(jnp.uint32)
        h3 = lax.rem(mm(t2) + t1, T)
        pre = t1 * sel2 + h3 * (jnp.uint32(1) - sel2)
        h = lax.rem(mm(pre) + t0, T)
        idx = h.astype(jnp.int32) + j * jnp.int32(_TABLE)
        r = lax.div(pos, jnp.int32(_CH))
        co = lax.rem(pos, jnp.int32(_CH))
        idxbuf[r, pl.ds(co, 16)] = idx
        return pos + 16

    lax.fori_loop(0, _S // 16, hash_body, jnp.int32(0))

    base = wid * _S

    def g_copy(t, bufi):
        return pltpu.make_async_copy(
            table_hbm.at[idxbuf.at[jnp.int32(t)]],
            rows.at[jnp.int32(bufi)], gs[bufi])

    def w_copy(t, bufi):
        return pltpu.make_async_copy(
            rows.at[jnp.int32(bufi)],
            out_hbm.at[pl.ds(base + t * _CH, _CH)], ws[bufi])

    for t in range(_NBUF):
        g_copy(t, t).start()
    for t in range(_NCH):
        bufi = t % _NBUF
        g_copy(t, bufi).wait()
        w_copy(t, bufi).start()
        if t + _NBUF < _NCH:
            w_copy(t, bufi).wait()
            g_copy(t + _NBUF, bufi).start()
    for t in range(_NCH - _NBUF, _NCH):
        w_copy(t, t % _NBUF).wait()


def _sc_gather(tok_flat, table_flat):
    mesh = plsc.VectorSubcoreMesh(
        core_axis_name="c", subcore_axis_name="s", num_cores=2,
        num_subcores=16)
    f = pl.kernel(
        _sc_body,
        out_type=jax.ShapeDtypeStruct((_NPARTS * _ROWS, _EMB), jnp.float32),
        mesh=mesh,
        scratch_types=[
            pltpu.VMEM((16 + _S,), jnp.int32),
            pltpu.VMEM((_NCH, _CH), jnp.int32),
            pltpu.VMEM((_NBUF, _CH, _EMB), jnp.float32),
            pltpu.SemaphoreType.DMA,
            pltpu.SemaphoreType.DMA,
            pltpu.SemaphoreType.DMA,
            pltpu.SemaphoreType.DMA,
            pltpu.SemaphoreType.DMA,
            pltpu.SemaphoreType.DMA,
        ],
    )
    return f(tok_flat, table_flat)


_CHUNK = 512
_NSTEP = _ROWS // _CHUNK
_SPC = _S // _CHUNK


def _tc_body(parts_ref, hid_ref, mn_g_ref, mn_b_ref, qn_g_ref, qn_b_ref,
             wq_ref, wk_ref, wv_ref, wo_ref, bo_ref, on_g_ref, on_b_ref,
             cw_ref, cb_ref, mv_ref, w_ref, carry_ref, ext_ref):
    c = pl.program_id(0)
    cmod = lax.rem(c, jnp.int32(_SPC))
    coff = cmod * _CHUNK
    row = coff + lax.broadcasted_iota(jnp.int32, (_CHUNK, 1), 0)
    m2 = (row >= 1).astype(jnp.float32)
    m3 = (row >= 2).astype(jnp.float32)

    xs = []
    s1 = jnp.zeros((_CHUNK, 1), jnp.float32)
    s2 = jnp.zeros((_CHUNK, 1), jnp.float32)
    for j in range(_NPARTS):
        xj = parts_ref[j] * (m2 if j < _NUM_HEADS else m3)
        xs.append(xj)
        s1 = s1 + jnp.sum(xj, axis=1, keepdims=True)
        s2 = s2 + jnp.sum(xj * xj, axis=1, keepdims=True)
    mean = s1 / _CONCAT
    var = s2 / _CONCAT - mean * mean
    rstd = lax.rsqrt(var + 1e-5)

    k = jnp.zeros((_CHUNK, _EMB), jnp.float32)
    v = jnp.zeros((_CHUNK, _EMB), jnp.float32)
    for j in range(_NPARTS):
        xn = (xs[j] - mean) * rstd * mn_g_ref[j][None, :] + mn_b_ref[j][None, :]
        xnb = xn.astype(jnp.bfloat16)
        k = k + jnp.dot(xnb, wk_ref[j].astype(jnp.bfloat16),
                        preferred_element_type=jnp.float32)
        v = v + jnp.dot(xnb, wv_ref[j].astype(jnp.bfloat16),
                        preferred_element_type=jnp.float32)

    h = hid_ref[...]
    hm = jnp.mean(h, axis=1, keepdims=True)
    hv = jnp.mean(h * h, axis=1, keepdims=True) - hm * hm
    hn = (h - hm) * lax.rsqrt(hv + 1e-5) * qn_g_ref[...] + qn_b_ref[...]
    q = jnp.dot(hn.astype(jnp.bfloat16), wq_ref[...].astype(jnp.bfloat16),
                preferred_element_type=jnp.float32)

    scores = jnp.sum(q * k, axis=1, keepdims=True) * (1.0 / 16.0)
    w = jax.nn.sigmoid(scores)

    o = jnp.dot((w * v).astype(jnp.bfloat16), wo_ref[...].astype(jnp.bfloat16),
                preferred_element_type=jnp.float32) + bo_ref[...]
    om = jnp.mean(o, axis=1, keepdims=True)
    ov = jnp.mean(o * o, axis=1, keepdims=True) - om * om
    out = (o - om) * lax.rsqrt(ov + 1e-5) * on_g_ref[...] + on_b_ref[...]

    @pl.when(cmod == 0)
    def _():
        carry_ref[...] = jnp.zeros((8, _EMB), jnp.float32)

    ext_ref[pl.ds(0, 8), :] = carry_ref[...]
    ext_ref[pl.ds(8, _CHUNK), :] = out
    conv = cb_ref[...] + cw_ref[3][None, :] * out
    for i in range(_KSIZE - 1):
        conv = conv + cw_ref[i][None, :] * ext_ref[pl.ds(5 + i, _CHUNK), :]
    carry_ref[...] = out[_CHUNK - 8:, :]

    mv_ref[...] = out + conv
    w_ref[...] = w


def _tc_dense(parts, hidden, mn_g, mn_b, qn_g, qn_b, Wq, Wk, Wv, Wo, bo,
              on_g, on_b, cw, cb):
    grid = (_NSTEP,)
    out_shapes = (
        jax.ShapeDtypeStruct((_ROWS, _EMB), jnp.float32),
        jax.ShapeDtypeStruct((_ROWS, 1), jnp.float32),
    )
    full = lambda *s: pl.BlockSpec(s, lambda c: tuple(c * 0 for _ in s))
    return pl.pallas_call(
        _tc_body,
        grid=grid,
        in_specs=[
            pl.BlockSpec((_NPARTS, _CHUNK, _EMB), lambda c: (c * 0, c, c * 0)),
            pl.BlockSpec((_CHUNK, _HID), lambda c: (c, c * 0)),
            full(_NPARTS, _EMB),
            full(_NPARTS, _EMB),
            full(1, _HID),
            full(1, _HID),
            full(_HID, _EMB),
            full(_NPARTS, _EMB, _EMB),
            full(_NPARTS, _EMB, _EMB),
            full(_EMB, _EMB),
            full(1, _EMB),
            full(1, _EMB),
            full(1, _EMB),
            full(_KSIZE, _EMB),
            full(1, _EMB),
        ],
        out_specs=(
            pl.BlockSpec((_CHUNK, _EMB), lambda c: (c, c * 0)),
            pl.BlockSpec((_CHUNK, 1), lambda c: (c, c * 0)),
        ),
        out_shape=out_shapes,
        scratch_shapes=[
            pltpu.VMEM((8, _EMB), jnp.float32),
            pltpu.VMEM((8 + _CHUNK, _EMB), jnp.float32),
        ],
        compiler_params=pltpu.CompilerParams(
            dimension_semantics=("arbitrary",)),
    )(parts, hidden, mn_g, mn_b, qn_g, qn_b, Wq, Wk, Wv, Wo, bo, on_g,
      on_b, cw, cb)


def kernel(hidden_states, token_ids, tables, qn_g, qn_b, mn_g, mn_b, Wq,
           Wk, Wv, Wo, bo, on_g, on_b, conv_w, conv_b):
    tok_flat = token_ids.astype(jnp.int32).reshape(_ROWS)
    table_flat = tables.reshape(_NPARTS * _TABLE, _EMB)
    parts = _sc_gather(tok_flat, table_flat).reshape(_NPARTS, _ROWS, _EMB)

    hidden = hidden_states.reshape(_ROWS, _HID)
    cw = jnp.transpose(conv_w[:, 0, :], (1, 0))
    mv, w = _tc_dense(
        parts, hidden,
        mn_g.reshape(_NPARTS, _EMB), mn_b.reshape(_NPARTS, _EMB),
        qn_g.reshape(1, _HID), qn_b.reshape(1, _HID),
        Wq, Wk.reshape(_NPARTS, _EMB, _EMB), Wv.reshape(_NPARTS, _EMB, _EMB),
        Wo, bo.reshape(1, _EMB), on_g.reshape(1, _EMB), on_b.reshape(1, _EMB),
        cw, conv_b.reshape(1, _EMB))
    return mv.reshape(_B, _S, _EMB), w.reshape(_B, _S)


# speedup vs baseline: 4.7484x; 1.0700x over previous
import functools

import numpy as np
import jax
import jax.numpy as jnp
from jax import lax
from jax.experimental import pallas as pl
from jax.experimental.pallas import tpu as pltpu
from jax.experimental.pallas import tpu_sc as plsc

_NUM_HEADS = 4
_TABLE = 50000
_EMB = 256
_HID = 1536
_KSIZE = 4
_B, _S = 4, 2048
_NPARTS = 8
_ROWS = _B * _S
_CONCAT = _EMB * _NPARTS

_mrng = np.random.RandomState(42)
_MULTS_NP = [_mrng.randint(2, 2 ** 31, size=_NUM_HEADS) for _ in range(2)]
_M = [int(_MULTS_NP[j // _NUM_HEADS][j % _NUM_HEADS]) for j in range(_NPARTS)]
_R16 = (1 << 16) % _TABLE

_NW = 32
_CH = 128
_NCH = _S // _CH
_NBUF = 3


def _sc_body(tok_hbm, table_hbm, out_hbm, tbuf, idxbuf, rows, zbuf,
             g0, g1, g2, w0, w1, w2):
    gs = (g0, g1, g2)
    ws = (w0, w1, w2)
    cid = lax.axis_index("c")
    sid = lax.axis_index("s")
    wid = (sid * 2 + cid).astype(jnp.int32)
    j = lax.div(wid, jnp.int32(_B))
    b = lax.rem(wid, jnp.int32(_B))

    tbuf[pl.ds(0, 16)] = jnp.zeros((16,), jnp.int32)
    pltpu.sync_copy(tok_hbm.at[pl.ds(b * _S, _S)], tbuf.at[pl.ds(16, _S)])

    m_lo = jnp.uint32(0)
    m_hi = jnp.uint32(0)
    for jj in range(_NPARTS):
        sel = j == jj
        m_lo = jnp.where(sel, jnp.uint32(_M[jj] & 0xFFFF), m_lo)
        m_hi = jnp.where(sel, jnp.uint32(_M[jj] >> 16), m_hi)
    sel2 = jnp.where(j < _NUM_HEADS, jnp.uint32(1), jnp.uint32(0))

    T = jnp.uint32(_TABLE)
    R = jnp.uint32(_R16)

    def mm(h):
        a = lax.rem(h * m_lo, T)
        c = lax.rem(h * m_hi, T)
        return lax.rem(c * R + a, T)

    def hash_body(_, pos):
        t0 = tbuf[pl.ds(16 + pos, 16)].astype(jnp.uint32)
        t1 = tbuf[pl.ds(15 + pos, 16)].astype(jnp.uint32)
        t2 = tbuf[pl.ds(14 + pos, 16)].astype(jnp.uint32)
        h3 = lax.rem(mm(t2) + t1, T)
        pre = t1 * sel2 + h3 * (jnp.uint32(1) - sel2)
        h = lax.rem(mm(pre) + t0, T)
        idx = h.astype(jnp.int32) + j * jnp.int32(_TABLE)
        r = lax.div(pos, jnp.int32(_CH))
        co = lax.rem(pos, jnp.int32(_CH))
        idxbuf[r, pl.ds(co, 16)] = idx
        return pos + 16

    lax.fori_loop(0, _S // 16, hash_body, jnp.int32(0))

    base = wid * _S

    def g_copy(t, bufi):
        return pltpu.make_async_copy(
            table_hbm.at[idxbuf.at[jnp.int32(t)]],
            rows.at[jnp.int32(bufi)], gs[bufi])

    def w_copy(t, bufi):
        return pltpu.make_async_copy(
            rows.at[jnp.int32(bufi)],
            out_hbm.at[pl.ds(base + t * _CH, _CH)], ws[bufi])

    for t in range(_NBUF):
        g_copy(t, t).start()
    for t in range(_NCH):
        bufi = t % _NBUF
        g_copy(t, bufi).wait()
        w_copy(t, bufi).start()
        if t + _NBUF < _NCH:
            w_copy(t, bufi).wait()
            g_copy(t + _NBUF, bufi).start()
    for t in range(_NCH - _NBUF, _NCH):
        w_copy(t, t % _NBUF).wait()

    for r in range(2):
        for i in range(_EMB // 16):
            zbuf[r, pl.ds(i * 16, 16)] = jnp.zeros((16,), jnp.float32)
    pltpu.sync_copy(zbuf.at[pl.ds(0, 1)], out_hbm.at[pl.ds(base, 1)])

    @pl.when(j >= _NUM_HEADS)
    def _():
        pltpu.sync_copy(zbuf.at[pl.ds(1, 1)],
                        out_hbm.at[pl.ds(base + 1, 1)])


def _sc_gather(tok_flat, table_flat):
    mesh = plsc.VectorSubcoreMesh(
        core_axis_name="c", subcore_axis_name="s", num_cores=2,
        num_subcores=16)
    f = pl.kernel(
        _sc_body,
        out_type=jax.ShapeDtypeStruct((_NPARTS * _ROWS, _EMB), jnp.float32),
        mesh=mesh,
        scratch_types=[
            pltpu.VMEM((16 + _S,), jnp.int32),
            pltpu.VMEM((_NCH, _CH), jnp.int32),
            pltpu.VMEM((_NBUF, _CH, _EMB), jnp.float32),
            pltpu.VMEM((2, _EMB), jnp.float32),
            pltpu.SemaphoreType.DMA,
            pltpu.SemaphoreType.DMA,
            pltpu.SemaphoreType.DMA,
            pltpu.SemaphoreType.DMA,
            pltpu.SemaphoreType.DMA,
            pltpu.SemaphoreType.DMA,
        ],
    )
    return f(tok_flat, table_flat)


_CHUNK = 512
_NSTEP = _ROWS // _CHUNK
_SPC = _S // _CHUNK


def _tc_body(parts_ref, hid_ref, ck_ref, bk_ref, cv_ref, bv_ref, cq_ref,
             bq_ref, wq_ref, wk_ref, wv_ref, wo_ref, bo_ref, on_g_ref,
             on_b_ref, cw_ref, cb_ref, mv_ref, w_ref, carry_ref, ext_ref):
    c = pl.program_id(0)
    cmod = lax.rem(c, jnp.int32(_SPC))

    ps = [parts_ref[j] for j in range(_NPARTS)]
    acc = ps[0]
    acc2 = ps[0] * ps[0]
    for j in range(1, _NPARTS):
        acc = acc + ps[j]
        acc2 = acc2 + ps[j] * ps[j]
    s1 = jnp.sum(acc, axis=1, keepdims=True)
    s2 = jnp.sum(acc2, axis=1, keepdims=True)
    mean = s1 * (1.0 / _CONCAT)
    var = s2 * (1.0 / _CONCAT) - mean * mean
    rstd = lax.rsqrt(var + 1e-5)

    kraw = jnp.dot(ps[0].astype(jnp.bfloat16), wk_ref[0],
                   preferred_element_type=jnp.float32)
    vraw = jnp.dot(ps[0].astype(jnp.bfloat16), wv_ref[0],
                   preferred_element_type=jnp.float32)
    for j in range(1, _NPARTS):
        pb = ps[j].astype(jnp.bfloat16)
        kraw = kraw + jnp.dot(pb, wk_ref[j],
                              preferred_element_type=jnp.float32)
        vraw = vraw + jnp.dot(pb, wv_ref[j],
                              preferred_element_type=jnp.float32)
    rm = rstd * mean
    k = kraw * rstd - rm * ck_ref[...] + bk_ref[...]
    v = vraw * rstd - rm * cv_ref[...] + bv_ref[...]

    h = hid_ref[...]
    s1h = jnp.sum(h, axis=1, keepdims=True)
    s2h = jnp.sum(h * h, axis=1, keepdims=True)
    hm = s1h * (1.0 / _HID)
    hv = s2h * (1.0 / _HID) - hm * hm
    hr = lax.rsqrt(hv + 1e-5)
    qraw = jnp.dot(h.astype(jnp.bfloat16), wq_ref[...],
                   preferred_element_type=jnp.float32)
    q = qraw * hr - (hr * hm) * cq_ref[...] + bq_ref[...]

    scores = jnp.sum(q * k, axis=1, keepdims=True) * (1.0 / 16.0)
    w = jax.nn.sigmoid(scores)

    o = jnp.dot((w * v).astype(jnp.bfloat16), wo_ref[...],
                preferred_element_type=jnp.float32) + bo_ref[...]
    om = jnp.mean(o, axis=1, keepdims=True)
    ov = jnp.mean(o * o, axis=1, keepdims=True) - om * om
    out = (o - om) * lax.rsqrt(ov + 1e-5) * on_g_ref[...] + on_b_ref[...]

    @pl.when(cmod == 0)
    def _():
        carry_ref[...] = jnp.zeros((8, _EMB), jnp.float32)

    ext_ref[pl.ds(0, 8), :] = carry_ref[...]
    ext_ref[pl.ds(8, _CHUNK), :] = out
    conv = cb_ref[...] + cw_ref[3][None, :] * out
    for i in range(_KSIZE - 1):
        conv = conv + cw_ref[i][None, :] * ext_ref[pl.ds(5 + i, _CHUNK), :]
    carry_ref[...] = out[_CHUNK - 8:, :]

    mv_ref[...] = out + conv
    w_ref[...] = w


def _tc_dense(parts, hidden, ck, bk, cv, bv, cq, bq, Wq, Wk, Wv, Wo, bo,
              on_g, on_b, cw, cb):
    grid = (_NSTEP,)
    out_shapes = (
        jax.ShapeDtypeStruct((_ROWS, _EMB), jnp.float32),
        jax.ShapeDtypeStruct((_ROWS, 1), jnp.float32),
    )
    full = lambda *s: pl.BlockSpec(s, lambda c: tuple(c * 0 for _ in s))
    return pl.pallas_call(
        _tc_body,
        grid=grid,
        in_specs=[
            pl.BlockSpec((_NPARTS, _CHUNK, _EMB), lambda c: (c * 0, c, c * 0)),
            pl.BlockSpec((_CHUNK, _HID), lambda c: (c, c * 0)),
            full(1, _EMB),
            full(1, _EMB),
            full(1, _EMB),
            full(1, _EMB),
            full(1, _EMB),
            full(1, _EMB),
            full(_HID, _EMB),
            full(_NPARTS, _EMB, _EMB),
            full(_NPARTS, _EMB, _EMB),
            full(_EMB, _EMB),
            full(1, _EMB),
            full(1, _EMB),
            full(1, _EMB),
            full(_KSIZE, _EMB),
            full(1, _EMB),
        ],
        out_specs=(
            pl.BlockSpec((_CHUNK, _EMB), lambda c: (c, c * 0)),
            pl.BlockSpec((_CHUNK, 1), lambda c: (c, c * 0)),
        ),
        out_shape=out_shapes,
        scratch_shapes=[
            pltpu.VMEM((8, _EMB), jnp.float32),
            pltpu.VMEM((8 + _CHUNK, _EMB), jnp.float32),
        ],
        compiler_params=pltpu.CompilerParams(
            dimension_semantics=("arbitrary",)),
    )(parts, hidden, ck, bk, cv, bv, cq, bq, Wq, Wk, Wv, Wo, bo, on_g,
      on_b, cw, cb)


def kernel(hidden_states, token_ids, tables, qn_g, qn_b, mn_g, mn_b, Wq,
           Wk, Wv, Wo, bo, on_g, on_b, conv_w, conv_b):
    tok_flat = token_ids.astype(jnp.int32).reshape(_ROWS)
    table_flat = tables.reshape(_NPARTS * _TABLE, _EMB)
    parts = _sc_gather(tok_flat, table_flat).reshape(_NPARTS, _ROWS, _EMB)

    hidden = hidden_states.reshape(_ROWS, _HID)
    cw = jnp.transpose(conv_w[:, 0, :], (1, 0))

    Wkg = Wk * mn_g[:, None]
    Wvg = Wv * mn_g[:, None]
    Wqg = Wq * qn_g[:, None]
    ck = jnp.sum(Wkg, axis=0).reshape(1, _EMB)
    cv = jnp.sum(Wvg, axis=0).reshape(1, _EMB)
    cq = jnp.sum(Wqg, axis=0).reshape(1, _EMB)
    bk = (mn_b @ Wk).reshape(1, _EMB)
    bv = (mn_b @ Wv).reshape(1, _EMB)
    bq = (qn_b @ Wq).reshape(1, _EMB)

    mv, w = _tc_dense(
        parts, hidden, ck, bk, cv, bv, cq, bq,
        Wqg.astype(jnp.bfloat16),
        Wkg.reshape(_NPARTS, _EMB, _EMB).astype(jnp.bfloat16),
        Wvg.reshape(_NPARTS, _EMB, _EMB).astype(jnp.bfloat16),
        Wo.astype(jnp.bfloat16),
        bo.reshape(1, _EMB), on_g.reshape(1, _EMB), on_b.reshape(1, _EMB),
        cw, conv_b.reshape(1, _EMB))
    return mv.reshape(_B, _S, _EMB), w.reshape(_B, _S)
